# baseline (speedup 1.0000x reference)
import jax
import jax.numpy as jnp
from jax import lax
from jax.experimental import pallas as pl
from jax.experimental.pallas import tpu as pltpu
from jax.experimental.pallas import tpu_sc as plsc

_H = 128
_B = 80
_G = _B // 16


def _prep_body(emb_ref, d_ref, scale_ref, a_ref, b_ref):
    x = emb_ref[...]
    ss = jnp.sum(x * x, axis=1, keepdims=True)
    norm = jnp.maximum(jnp.sqrt(ss), 1e-12)
    e = x / norm
    b_ref[...] = e
    a_ref[...] = e * (d_ref[...] * scale_ref[0])[None, :]


def _prep(emb, d, scale):
    return pl.pallas_call(
        _prep_body,
        out_shape=(
            jax.ShapeDtypeStruct(emb.shape, jnp.float32),
            jax.ShapeDtypeStruct(emb.shape, jnp.float32),
        ),
    )(emb, d, scale)


def _edge_body(a_hbm, b_hbm, src_hbm, dst_hbm, out_hbm,
               stage, si0, di0, si1, di1, ov0, ov1, tspa, ar0, br0, ar1, br1,
               ssi0, ssi1, sa0, sb0, sa1, sb1, so0, so1):
    ep = out_hbm.shape[0] // 32
    nb = ep // _B
    n_nodes = a_hbm.shape[0]
    nrows = (n_nodes // 16) // 8 * 8
    rem = n_nodes - 16 * nrows
    sub = lax.axis_index("s")
    wid = sub * 2 + lax.axis_index("c")
    base = pl.multiple_of(wid * ep, 8)
    lane = lax.iota(jnp.int32, 16)

    roff = pl.multiple_of(sub * nrows, 8)
    pltpu.sync_copy(a_hbm.at[pl.ds(roff, nrows)], tspa.at[pl.ds(roff, nrows)])
    if rem:
        @pl.when(sub == 0)
        def _():
            pltpu.sync_copy(a_hbm.at[pl.ds(16 * nrows, rem)],
                            tspa.at[pl.ds(16 * nrows, rem)])
    plsc.subcore_barrier()

    bufs = ((si0, di0, ssi0, ar0, br0, sa0, sb0, ov0, so0),
            (si1, di1, ssi1, ar1, br1, sa1, sb1, ov1, so1))

    def fetch(ib, buf):
        si, di, ssi = buf[0], buf[1], buf[2]
        off = pl.multiple_of(base + ib * _B, 8)
        pltpu.async_copy(src_hbm.at[pl.ds(off, _B)], si, ssi)
        pltpu.async_copy(dst_hbm.at[pl.ds(off, _B)], di, ssi)

    def wait_idx(buf):
        si, di, ssi = buf[0], buf[1], buf[2]
        pltpu.make_async_copy(src_hbm.at[pl.ds(0, _B)], si, ssi).wait()
        pltpu.make_async_copy(dst_hbm.at[pl.ds(0, _B)], di, ssi).wait()

    def gather(buf):
        si, di, ar, br, sa, sb = buf[0], buf[1], buf[3], buf[4], buf[5], buf[6]
        pltpu.async_copy(a_hbm.at[si], ar, sa)
        pltpu.async_copy(b_hbm.at[di], br, sb)

    def wait_gather(buf):
        si, di, ar, br, sa, sb = buf[0], buf[1], buf[3], buf[4], buf[5], buf[6]
        pltpu.make_async_copy(a_hbm.at[si], ar, sa).wait()
        pltpu.make_async_copy(b_hbm.at[di], br, sb).wait()

    def wait_out(buf):
        ov, so = buf[7], buf[8]
        pltpu.make_async_copy(ov, out_hbm.at[pl.ds(base, _B)], so).wait()

    def tree_sum(vs):
        while len(vs) > 1:
            vs = [a + b for a, b in zip(vs[::2], vs[1::2])] + \
                 ([vs[-1]] if len(vs) % 2 else [])
        return vs[0]

    def compute(ib, buf):
        ar, br, ov, so = buf[3], buf[4], buf[7], buf[8]

        def group_body(g, _):
            e0 = g * 16
            for j in range(16):
                e = e0 + j
                prods = []
                for c in range(_H // 16):
                    va = ar[e, pl.ds(c * 16, 16)]
                    vb = br[e, pl.ds(c * 16, 16)]
                    prods.append(va * vb)
                acc = tree_sum(prods)
                plsc.store_scatter(stage, [lane * 16 + j], acc)
            tot = tree_sum([stage[pl.ds(l * 16, 16)] for l in range(16)])
            ov[pl.ds(pl.multiple_of(g * 16, 16), 16)] = tot
            return 0

        lax.fori_loop(0, _G, group_body, 0)
        off = pl.multiple_of(base + ib * _B, 8)
        pltpu.async_copy(ov, out_hbm.at[pl.ds(off, _B)], so)

    fetch(0, bufs[0])
    fetch(1, bufs[1])
    wait_idx(bufs[0])
    gather(bufs[0])

    def pair_body(i2, _):
        ib = i2 * 2
        wait_idx(bufs[1])
        gather(bufs[1])

        wait_gather(bufs[0])

        @pl.when(i2 > 0)
        def _():
            wait_out(bufs[0])
        compute(ib, bufs[0])
        fetch(ib + 2, bufs[0])

        wait_gather(bufs[1])

        @pl.when(i2 > 0)
        def _():
            wait_out(bufs[1])
        compute(ib + 1, bufs[1])

        @pl.when(ib + 3 < nb)
        def _():
            fetch(ib + 3, bufs[1])

        wait_idx(bufs[0])
        gather(bufs[0])
        return 0

    lax.fori_loop(0, (nb - 1) // 2, pair_body, 0)
    wait_gather(bufs[0])
    wait_out(bufs[0])
    compute(nb - 1, bufs[0])
    wait_out(bufs[0])
    wait_out(bufs[1])


def _edge_dot(a, b, src, dst):
    n_edges = src.shape[0]
    mesh = plsc.VectorSubcoreMesh(core_axis_name="c", subcore_axis_name="s")
    return pl.kernel(
        _edge_body,
        out_type=jax.ShapeDtypeStruct((n_edges,), jnp.float32),
        mesh=mesh,
        compiler_params=pltpu.CompilerParams(needs_layout_passes=False),
        scratch_types=[
            pltpu.VMEM((256,), jnp.float32),
            pltpu.VMEM((_B,), jnp.int32),
            pltpu.VMEM((_B,), jnp.int32),
            pltpu.VMEM((_B,), jnp.int32),
            pltpu.VMEM((_B,), jnp.int32),
            pltpu.VMEM((_B,), jnp.float32),
            pltpu.VMEM((_B,), jnp.float32),
            pltpu.VMEM_SHARED(a.shape, jnp.float32),
            pltpu.VMEM((_B, _H), jnp.float32),
            pltpu.VMEM((_B, _H), jnp.float32),
            pltpu.VMEM((_B, _H), jnp.float32),
            pltpu.VMEM((_B, _H), jnp.float32),
            pltpu.SemaphoreType.DMA,
            pltpu.SemaphoreType.DMA,
            pltpu.SemaphoreType.DMA,
            pltpu.SemaphoreType.DMA,
            pltpu.SemaphoreType.DMA,
            pltpu.SemaphoreType.DMA,
            pltpu.SemaphoreType.DMA,
            pltpu.SemaphoreType.DMA,
        ],
    )(a, b, src, dst)


def kernel(emb, edge_index, d, scale):
    src = edge_index[0].astype(jnp.int32)
    dst = edge_index[1].astype(jnp.int32)
    a, b = _prep(emb, d, scale)
    out = _edge_dot(a, b, src, dst)
    return out.reshape(-1, 1)


# speedup vs baseline: 1.3643x; 1.3643x over previous
import jax
import jax.numpy as jnp
from jax import lax
from jax.experimental import pallas as pl
from jax.experimental.pallas import tpu as pltpu
from jax.experimental.pallas import tpu_sc as plsc

_H = 128
_B = 80
_G = _B // 16


def _prep_body(emb_ref, d_ref, scale_ref, a_ref, b_ref):
    x = emb_ref[...]
    ss = jnp.sum(x * x, axis=1, keepdims=True)
    norm = jnp.maximum(jnp.sqrt(ss), 1e-12)
    e = x / norm
    b_ref[...] = e
    a_ref[...] = e * (d_ref[...] * scale_ref[0])[None, :]


def _prep(emb, d, scale):
    return pl.pallas_call(
        _prep_body,
        out_shape=(
            jax.ShapeDtypeStruct(emb.shape, jnp.float32),
            jax.ShapeDtypeStruct(emb.shape, jnp.float32),
        ),
    )(emb, d, scale)


def _edge_body(a_hbm, b_hbm, src_hbm, dst_hbm, out_hbm,
               stage, si0, di0, si1, di1, si2, di2, si3, di3, ov0, ov1,
               tspa, ar0, br0, ar1, br1,
               ssi0, ssi1, ssi2, ssi3, sa0, sb0, sa1, sb1, so0, so1):
    ep = out_hbm.shape[0] // 32
    nb = ep // _B
    n_nodes = a_hbm.shape[0]
    nrows = (n_nodes // 16) // 8 * 8
    rem = n_nodes - 16 * nrows
    sub = lax.axis_index("s")
    wid = sub * 2 + lax.axis_index("c")
    base = pl.multiple_of(wid * ep, 8)
    lane = lax.iota(jnp.int32, 16)

    roff = pl.multiple_of(sub * nrows, 8)
    pltpu.sync_copy(a_hbm.at[pl.ds(roff, nrows)], tspa.at[pl.ds(roff, nrows)])
    if rem:
        @pl.when(sub == 0)
        def _():
            pltpu.sync_copy(a_hbm.at[pl.ds(16 * nrows, rem)],
                            tspa.at[pl.ds(16 * nrows, rem)])
    plsc.subcore_barrier()

    iq = ((si0, di0, ssi0), (si1, di1, ssi1),
          (si2, di2, ssi2), (si3, di3, ssi3))
    bufs = ((ar0, br0, sa0, sb0, ov0, so0),
            (ar1, br1, sa1, sb1, ov1, so1))

    def fetch(ib, q):
        si, di, ssi = iq[q]
        off = pl.multiple_of(base + ib * _B, 8)
        pltpu.async_copy(src_hbm.at[pl.ds(off, _B)], si, ssi)
        pltpu.async_copy(dst_hbm.at[pl.ds(off, _B)], di, ssi)

    def wait_idx(q):
        si, di, ssi = iq[q]
        pltpu.make_async_copy(src_hbm.at[pl.ds(0, _B)], si, ssi).wait()
        pltpu.make_async_copy(dst_hbm.at[pl.ds(0, _B)], di, ssi).wait()

    def gather(q, buf):
        si, di = iq[q][0], iq[q][1]
        ar, br, sa, sb = buf[0], buf[1], buf[2], buf[3]
        pltpu.async_copy(tspa.at[si], ar, sa)
        pltpu.async_copy(b_hbm.at[di], br, sb)

    def wait_gather(q, buf):
        si, di = iq[q][0], iq[q][1]
        ar, br, sa, sb = buf[0], buf[1], buf[2], buf[3]
        pltpu.make_async_copy(tspa.at[si], ar, sa).wait()
        pltpu.make_async_copy(b_hbm.at[di], br, sb).wait()

    def wait_out(buf):
        ov, so = buf[4], buf[5]
        pltpu.make_async_copy(ov, out_hbm.at[pl.ds(base, _B)], so).wait()

    def tree_sum(vs):
        while len(vs) > 1:
            vs = [a + b for a, b in zip(vs[::2], vs[1::2])] + \
                 ([vs[-1]] if len(vs) % 2 else [])
        return vs[0]

    def compute(ib, buf):
        ar, br, ov, so = buf[0], buf[1], buf[4], buf[5]

        def group_body(g, _):
            e0 = g * 16
            for j in range(16):
                e = e0 + j
                prods = []
                for c in range(_H // 16):
                    va = ar[e, pl.ds(c * 16, 16)]
                    vb = br[e, pl.ds(c * 16, 16)]
                    prods.append(va * vb)
                acc = tree_sum(prods)
                plsc.store_scatter(stage, [lane * 16 + j], acc)
            tot = tree_sum([stage[pl.ds(l * 16, 16)] for l in range(16)])
            ov[pl.ds(pl.multiple_of(g * 16, 16), 16)] = tot
            return 0

        lax.fori_loop(0, _G, group_body, 0)
        off = pl.multiple_of(base + ib * _B, 8)
        pltpu.async_copy(ov, out_hbm.at[pl.ds(off, _B)], so)

    for q in range(4):
        fetch(q, q)
    wait_idx(0)
    gather(0, bufs[0])

    def pair(i2, qa):
        i2 = jnp.int32(i2)
        ib = i2 * 2
        qb = qa + 1
        qc = (qa + 2) % 4
        wait_idx(qb)
        gather(qb, bufs[1])

        wait_gather(qa, bufs[0])

        @pl.when(i2 > 0)
        def _():
            wait_out(bufs[0])
        compute(ib, bufs[0])

        @pl.when(ib + 4 < nb)
        def _():
            fetch(ib + 4, qa)

        wait_idx(qc)
        gather(qc, bufs[0])

        wait_gather(qb, bufs[1])

        @pl.when(i2 > 0)
        def _():
            wait_out(bufs[1])
        compute(ib + 1, bufs[1])

        @pl.when(ib + 5 < nb)
        def _():
            fetch(ib + 5, qb)

    pairs = (nb - 1) // 2

    def quad_body(i4, _):
        pair(2 * i4, 0)
        pair(2 * i4 + 1, 2)
        return 0

    lax.fori_loop(0, pairs // 2, quad_body, 0)
    if pairs % 2:
        pair(pairs - 1, ((pairs - 1) % 2) * 2)
    wait_gather(((nb - 1) // 2 % 2) * 2, bufs[0])
    wait_out(bufs[0])
    compute(nb - 1, bufs[0])
    wait_out(bufs[0])
    wait_out(bufs[1])


def _edge_dot(a, b, src, dst):
    n_edges = src.shape[0]
    mesh = plsc.VectorSubcoreMesh(core_axis_name="c", subcore_axis_name="s")
    return pl.kernel(
        _edge_body,
        out_type=jax.ShapeDtypeStruct((n_edges,), jnp.float32),
        mesh=mesh,
        compiler_params=pltpu.CompilerParams(needs_layout_passes=False),
        scratch_types=(
            [pltpu.VMEM((256,), jnp.float32)]
            + [pltpu.VMEM((_B,), jnp.int32)] * 8
            + [pltpu.VMEM((_B,), jnp.float32)] * 2
            + [pltpu.VMEM_SHARED(a.shape, jnp.float32)]
            + [pltpu.VMEM((_B, _H), jnp.float32)] * 4
            + [pltpu.SemaphoreType.DMA] * 10
        ),
    )(a, b, src, dst)


def kernel(emb, edge_index, d, scale):
    src = edge_index[0].astype(jnp.int32)
    dst = edge_index[1].astype(jnp.int32)
    a, b = _prep(emb, d, scale)
    out = _edge_dot(a, b, src, dst)
    return out.reshape(-1, 1)


# speedup vs baseline: 1.4294x; 1.0477x over previous
import jax
import jax.numpy as jnp
from jax import lax
from jax.experimental import pallas as pl
from jax.experimental.pallas import tpu as pltpu
from jax.experimental.pallas import tpu_sc as plsc

_H = 128
_B = 80
_G = _B // 16


def _prep_body(emb_ref, d_ref, scale_ref, a_ref, b_ref):
    x = emb_ref[...]
    ss = jnp.sum(x * x, axis=1, keepdims=True)
    norm = jnp.maximum(jnp.sqrt(ss), 1e-12)
    e = x / norm
    b_ref[...] = e
    a_ref[...] = e * (d_ref[...] * scale_ref[0])[None, :]


def _prep(emb, d, scale):
    return pl.pallas_call(
        _prep_body,
        out_shape=(
            jax.ShapeDtypeStruct(emb.shape, jnp.float32),
            jax.ShapeDtypeStruct(emb.shape, jnp.float32),
        ),
    )(emb, d, scale)


def _edge_body(a_hbm, b_hbm, src_hbm, dst_hbm, out_hbm,
               stage, si0, di0, si1, di1, si2, di2, si3, di3, ov0, ov1,
               tspa, ar0, br0, ar1, br1,
               ssi0, ssi1, ssi2, ssi3, sa0, sb0, sa1, sb1, so0, so1):
    ep = out_hbm.shape[0] // 32
    nb = ep // _B
    n_nodes = a_hbm.shape[0]
    nrows = (n_nodes // 16) // 8 * 8
    rem = n_nodes - 16 * nrows
    sub = lax.axis_index("s")
    wid = sub * 2 + lax.axis_index("c")
    base = pl.multiple_of(wid * ep, 8)
    lane = lax.iota(jnp.int32, 16)

    roff = pl.multiple_of(sub * nrows, 8)
    pltpu.sync_copy(a_hbm.at[pl.ds(roff, nrows)], tspa.at[pl.ds(roff, nrows)])
    if rem:
        @pl.when(sub == 0)
        def _():
            pltpu.sync_copy(a_hbm.at[pl.ds(16 * nrows, rem)],
                            tspa.at[pl.ds(16 * nrows, rem)])
    plsc.subcore_barrier()

    iq = ((si0, di0, ssi0), (si1, di1, ssi1),
          (si2, di2, ssi2), (si3, di3, ssi3))
    bufs = ((ar0, br0, sa0, sb0, ov0, so0),
            (ar1, br1, sa1, sb1, ov1, so1))

    def fetch(ib, q):
        si, di, ssi = iq[q]
        off = pl.multiple_of(base + ib * _B, 8)
        pltpu.async_copy(src_hbm.at[pl.ds(off, _B)], si, ssi)
        pltpu.async_copy(dst_hbm.at[pl.ds(off, _B)], di, ssi)

    def wait_idx(q):
        si, di, ssi = iq[q]
        pltpu.make_async_copy(src_hbm.at[pl.ds(0, _B)], si, ssi).wait()
        pltpu.make_async_copy(dst_hbm.at[pl.ds(0, _B)], di, ssi).wait()

    def gather(q, buf):
        si, di = iq[q][0], iq[q][1]
        ar, br, sa, sb = buf[0], buf[1], buf[2], buf[3]
        pltpu.async_copy(tspa.at[si], ar, sa)
        pltpu.async_copy(b_hbm.at[di], br, sb)

    def wait_gather(q, buf):
        si, di = iq[q][0], iq[q][1]
        ar, br, sa, sb = buf[0], buf[1], buf[2], buf[3]
        pltpu.make_async_copy(tspa.at[si], ar, sa).wait()
        pltpu.make_async_copy(b_hbm.at[di], br, sb).wait()

    def wait_out(buf):
        ov, so = buf[4], buf[5]
        pltpu.make_async_copy(ov, out_hbm.at[pl.ds(base, _B)], so).wait()

    def tree_sum(vs):
        while len(vs) > 1:
            vs = [a + b for a, b in zip(vs[::2], vs[1::2])] + \
                 ([vs[-1]] if len(vs) % 2 else [])
        return vs[0]

    def compute(ib, buf):
        ar, br, ov, so = buf[0], buf[1], buf[4], buf[5]

        def group_body(g, _):
            e0 = g * 16
            for j in range(16):
                e = e0 + j
                accs = [None, None]
                for c in range(_H // 16):
                    va = ar[e, pl.ds(c * 16, 16)]
                    vb = br[e, pl.ds(c * 16, 16)]
                    p = va * vb
                    k = c % 2
                    accs[k] = p if accs[k] is None else accs[k] + p
                acc = accs[0] + accs[1]
                plsc.store_scatter(stage, [lane * 16 + j], acc)
            tot = tree_sum([stage[pl.ds(l * 16, 16)] for l in range(16)])
            ov[pl.ds(pl.multiple_of(g * 16, 16), 16)] = tot
            return 0

        lax.fori_loop(0, _G, group_body, 0)
        off = pl.multiple_of(base + ib * _B, 8)
        pltpu.async_copy(ov, out_hbm.at[pl.ds(off, _B)], so)

    for q in range(4):
        fetch(q, q)
    wait_idx(0)
    gather(0, bufs[0])

    def pair(i2, qa):
        i2 = jnp.int32(i2)
        ib = i2 * 2
        qb = qa + 1
        qc = (qa + 2) % 4
        wait_idx(qb)
        gather(qb, bufs[1])

        wait_gather(qa, bufs[0])

        @pl.when(i2 > 0)
        def _():
            wait_out(bufs[0])
        compute(ib, bufs[0])

        @pl.when(ib + 4 < nb)
        def _():
            fetch(ib + 4, qa)

        wait_idx(qc)
        gather(qc, bufs[0])

        wait_gather(qb, bufs[1])

        @pl.when(i2 > 0)
        def _():
            wait_out(bufs[1])
        compute(ib + 1, bufs[1])

        @pl.when(ib + 5 < nb)
        def _():
            fetch(ib + 5, qb)

    pairs = (nb - 1) // 2

    def quad_body(i4, _):
        pair(2 * i4, 0)
        pair(2 * i4 + 1, 2)
        return 0

    lax.fori_loop(0, pairs // 2, quad_body, 0)
    if pairs % 2:
        pair(pairs - 1, ((pairs - 1) % 2) * 2)
    wait_gather(((nb - 1) // 2 % 2) * 2, bufs[0])
    wait_out(bufs[0])
    compute(nb - 1, bufs[0])
    wait_out(bufs[0])
    wait_out(bufs[1])


def _edge_dot(a, b, src, dst):
    n_edges = src.shape[0]
    mesh = plsc.VectorSubcoreMesh(core_axis_name="c", subcore_axis_name="s")
    return pl.kernel(
        _edge_body,
        out_type=jax.ShapeDtypeStruct((n_edges,), jnp.float32),
        mesh=mesh,
        compiler_params=pltpu.CompilerParams(needs_layout_passes=False),
        scratch_types=(
            [pltpu.VMEM((256,), jnp.float32)]
            + [pltpu.VMEM((_B,), jnp.int32)] * 8
            + [pltpu.VMEM((_B,), jnp.float32)] * 2
            + [pltpu.VMEM_SHARED(a.shape, jnp.float32)]
            + [pltpu.VMEM((_B, _H), jnp.float32)] * 4
            + [pltpu.SemaphoreType.DMA] * 10
        ),
    )(a, b, src, dst)


def kernel(emb, edge_index, d, scale):
    src = edge_index[0].astype(jnp.int32)
    dst = edge_index[1].astype(jnp.int32)
    a, b = _prep(emb, d, scale)
    out = _edge_dot(a, b, src, dst)
    return out.reshape(-1, 1)
